# baseline (speedup 1.0000x reference)
import functools

import jax
import jax.numpy as jnp
from jax import lax
from jax.experimental import pallas as pl
from jax.experimental.pallas import tpu as pltpu
from jax.experimental.pallas import tpu_sc as plsc

SUB = 128
KB = 6


def _sc_info():
    try:
        info = plsc.get_sparse_core_info()
        return info.num_cores, info.num_subcores
    except Exception:
        return 2, 16


def _round_up(a, b):
    return (a + b - 1) // b * b


@functools.lru_cache(maxsize=None)
def _make_sc_agg(NC, NS, R2, NT, W, RPT, G):
    mesh = plsc.VectorSubcoreMesh(core_axis_name="c", subcore_axis_name="s", num_cores=NC, num_subcores=NS)
    NTS = NT // NS

    def body(ei_hbm, tab_hbm, zero_hbm, out_hbm, idx, rows, acc, gsem, ssem, isem):
        c = lax.axis_index("c")
        s = lax.axis_index("s")
        wid = s * NC + c
        pltpu.sync_copy(zero_hbm.at[pl.ds(s * NTS, NTS)], acc.at[pl.ds(s * NTS, NTS)])
        plsc.subcore_barrier()
        row0 = wid * RPT

        pltpu.sync_copy(ei_hbm.at[pl.ds(row0, KB)], idx.at[0])

        def group(g, carry):
            b = lax.rem(g, 2)
            bp = 1 - b
            for j in range(KB):
                pltpu.async_copy(tab_hbm.at[idx.at[b, j, 0]], rows.at[b, j], gsem)
            @pl.when(g >= 1)
            def _():
                for j in range(KB):
                    pltpu.make_async_copy(
                        rows.at[bp, j], acc.at[idx.at[bp, j, 1]], ssem
                    ).wait()
            @pl.when(g + 1 < G)
            def _():
                pltpu.async_copy(
                    ei_hbm.at[pl.ds(row0 + (g + 1) * KB, KB)], idx.at[bp], isem
                )
            for j in range(KB):
                pltpu.make_async_copy(tab_hbm.at[idx.at[b, j, 0]], rows.at[b, j], gsem).wait()
                pltpu.async_copy(rows.at[b, j], acc.at[idx.at[b, j, 1]], ssem, add=True)
            @pl.when(g + 1 < G)
            def _():
                pltpu.make_async_copy(
                    ei_hbm.at[pl.ds(row0 + (g + 1) * KB, KB)], idx.at[bp], isem
                ).wait()
            return carry

        lax.fori_loop(0, G, group, 0)
        bl = (G - 1) % 2
        for j in range(KB):
            pltpu.make_async_copy(rows.at[bl, j], acc.at[idx.at[bl, j, 1]], ssem).wait()
        plsc.subcore_barrier()
        pltpu.sync_copy(acc.at[pl.ds(s * NTS, NTS)], out_hbm.at[c, pl.ds(s * NTS, NTS)])

    return pl.kernel(
        body,
        out_type=jax.ShapeDtypeStruct((NC, NT, W), jnp.float32),
        mesh=mesh,
        compiler_params=pltpu.CompilerParams(use_tc_tiling_on_sc=False),
        scratch_types=[
            pltpu.VMEM((2, KB, 2, SUB), jnp.int32),
            pltpu.VMEM((2, KB, SUB, W), jnp.float32),
            pltpu.VMEM_SHARED((NT, W), jnp.float32),
            pltpu.SemaphoreType.DMA,
            pltpu.SemaphoreType.DMA,
            pltpu.SemaphoreType.DMA,
        ],
    )


@functools.lru_cache(maxsize=None)
def _make_sc_deg(NC, NS, R2, NT, RPT, G):
    mesh = plsc.VectorSubcoreMesh(core_axis_name="c", subcore_axis_name="s", num_cores=NC, num_subcores=NS)

    def body(dst_hbm, zero_hbm, out_hbm, idx, hist, isem):
        c = lax.axis_index("c")
        s = lax.axis_index("s")
        wid = s * NC + c
        pltpu.sync_copy(zero_hbm, hist)
        row0 = wid * RPT
        ones = jnp.ones((16,), jnp.float32)

        pltpu.sync_copy(dst_hbm.at[pl.ds(row0, KB)], idx.at[0])

        def group(g, carry):
            b = lax.rem(g, 2)
            bn = 1 - b
            @pl.when(g + 1 < G)
            def _():
                pltpu.async_copy(
                    dst_hbm.at[pl.ds(row0 + (g + 1) * KB, KB)], idx.at[bn], isem
                )
            for j in range(KB):
                for k in range(SUB // 16):
                    v = idx[b, j, pl.ds(k * 16, 16)]
                    plsc.addupdate_scatter(hist, [v], ones)
            @pl.when(g + 1 < G)
            def _():
                pltpu.make_async_copy(
                    dst_hbm.at[pl.ds(row0 + (g + 1) * KB, KB)], idx.at[bn], isem
                ).wait()
            return carry

        lax.fori_loop(0, G, group, 0)
        pltpu.sync_copy(hist, out_hbm.at[c, s])

    return pl.kernel(
        body,
        out_type=jax.ShapeDtypeStruct((NC, NS, NT), jnp.float32),
        mesh=mesh,
        compiler_params=pltpu.CompilerParams(
            use_tc_tiling_on_sc=False, needs_layout_passes=False
        ),
        scratch_types=[
            pltpu.VMEM((2, KB, SUB), jnp.int32),
            pltpu.VMEM((NT,), jnp.float32),
            pltpu.SemaphoreType.DMA,
        ],
    )


def _tc_prescale(degp, x_pad, BR):
    P, NT, _ = degp.shape
    Din = x_pad.shape[1]

    def body(degp_ref, x_ref, ys_ref, dinv_ref):
        d = jnp.sum(degp_ref[...], axis=0)
        di = lax.rsqrt(d + 1.0)
        dinv_ref[...] = di
        ys_ref[...] = x_ref[...] * di

    return pl.pallas_call(
        body,
        grid=(NT // BR,),
        in_specs=[
            pl.BlockSpec((P, BR, 1), lambda r: (0, r, 0)),
            pl.BlockSpec((BR, Din), lambda r: (r, 0)),
        ],
        out_specs=[
            pl.BlockSpec((BR, Din), lambda r: (r, 0)),
            pl.BlockSpec((BR, 1), lambda r: (r, 0)),
        ],
        out_shape=[
            jax.ShapeDtypeStruct((NT, Din), jnp.float32),
            jax.ShapeDtypeStruct((NT, 1), jnp.float32),
        ],
    )(degp, x_pad)


def _tc_mid(S1p, ys1, dinv, W1, b1, W2, BR):
    NC, NT, Din = S1p.shape
    Dh = W1.shape[1]
    Do = W2.shape[1]

    def body(sp_ref, ys_ref, di_ref, w1_ref, b1_ref, w2_ref, out_ref):
        S = ys_ref[...]
        for i in range(NC):
            S = S + sp_ref[i]
        agg = di_ref[...] * S
        h = jnp.dot(agg, w1_ref[...], preferred_element_type=jnp.float32) + b1_ref[...]
        h = jnp.maximum(h, 0.0)
        y2 = jnp.dot(h, w2_ref[...], preferred_element_type=jnp.float32)
        out_ref[...] = y2 * di_ref[...]

    return pl.pallas_call(
        body,
        grid=(NT // BR,),
        in_specs=[
            pl.BlockSpec((NC, BR, Din), lambda r: (0, r, 0)),
            pl.BlockSpec((BR, Din), lambda r: (r, 0)),
            pl.BlockSpec((BR, 1), lambda r: (r, 0)),
            pl.BlockSpec((Din, Dh), lambda r: (0, 0)),
            pl.BlockSpec((1, Dh), lambda r: (0, 0)),
            pl.BlockSpec((Dh, Do), lambda r: (0, 0)),
        ],
        out_specs=pl.BlockSpec((BR, Do), lambda r: (r, 0)),
        out_shape=jax.ShapeDtypeStruct((NT, Do), jnp.float32),
    )(S1p, ys1, dinv, W1, b1.reshape(1, Dh), W2)


def _tc_final(S2p, ys2, dinv, b2, BR):
    NC, NT, Do = S2p.shape

    def body(sp_ref, ys_ref, di_ref, b2_ref, out_ref):
        S = ys_ref[...]
        for i in range(NC):
            S = S + sp_ref[i]
        out_ref[...] = di_ref[...] * S + b2_ref[...]

    return pl.pallas_call(
        body,
        grid=(NT // BR,),
        in_specs=[
            pl.BlockSpec((NC, BR, Do), lambda r: (0, r, 0)),
            pl.BlockSpec((BR, Do), lambda r: (r, 0)),
            pl.BlockSpec((BR, 1), lambda r: (r, 0)),
            pl.BlockSpec((1, Do), lambda r: (0, 0)),
        ],
        out_specs=pl.BlockSpec((BR, Do), lambda r: (r, 0)),
        out_shape=jax.ShapeDtypeStruct((NT, Do), jnp.float32),
    )(S2p, ys2, dinv, b2.reshape(1, Do))


def kernel(x, edge_index, W1, b1, W2, b2):
    N, Din = x.shape
    E = edge_index.shape[1]
    Dh = W1.shape[1]
    Do = W2.shape[1]
    NC, NS = _sc_info()
    NW = NC * NS

    unit = NW * SUB * KB
    E_pad = _round_up(E, unit)
    EPT = E_pad // NW
    RPT = EPT // SUB
    R2 = E_pad // SUB
    G = EPT // (SUB * KB)
    BR = 1024
    NT = _round_up(N + 1, BR)

    src = edge_index[0]
    dst = edge_index[1]
    padv = jnp.full((E_pad - E,), N, jnp.int32)
    srcp = jnp.concatenate([src, padv]).reshape(R2, SUB)
    dstp = jnp.concatenate([dst, padv]).reshape(R2, SUB)
    ei2 = jnp.stack([srcp, dstp], axis=1)
    x_pad = jnp.zeros((NT, Din), jnp.float32).at[:N].set(x)

    zeroN = jnp.zeros((NT,), jnp.float32)
    zero1 = jnp.zeros((NT, Din), jnp.float32)
    zero2 = jnp.zeros((NT, Do), jnp.float32)

    degp = _make_sc_deg(NC, NS, R2, NT, RPT, G)(dstp, zeroN)
    ys1, dinv = _tc_prescale(degp.reshape(NC * NS, NT, 1), x_pad, BR)
    S1p = _make_sc_agg(NC, NS, R2, NT, Din, RPT, G)(ei2, ys1, zero1)
    ys2 = _tc_mid(S1p, ys1, dinv, W1, b1, W2, BR)
    S2p = _make_sc_agg(NC, NS, R2, NT, Do, RPT, G)(ei2, ys2, zero2)
    out = _tc_final(S2p, ys2, dinv, b2, BR)
    return out[:N]


# speedup vs baseline: 2.3806x; 2.3806x over previous
import functools

import jax
import jax.numpy as jnp
from jax import lax
from jax.experimental import pallas as pl
from jax.experimental.pallas import tpu as pltpu
from jax.experimental.pallas import tpu_sc as plsc

SUB = 128
KB = 6


def _sc_info():
    try:
        info = plsc.get_sparse_core_info()
        return info.num_cores, info.num_subcores
    except Exception:
        return 2, 16


def _round_up(a, b):
    return (a + b - 1) // b * b


@functools.lru_cache(maxsize=None)
def _make_sc_agg(NC, NS, R2, NT, W, RPT, G):
    mesh = plsc.VectorSubcoreMesh(core_axis_name="c", subcore_axis_name="s", num_cores=NC, num_subcores=NS)
    NTS = NT // NS

    def body(ei_hbm, tab_hbm, zero_hbm, out_hbm, idx, rows, acc, gsem, ssem, isem):
        c = lax.axis_index("c")
        s = lax.axis_index("s")
        wid = s * NC + c
        pltpu.sync_copy(zero_hbm.at[pl.ds(s * NTS, NTS)], acc.at[pl.ds(s * NTS, NTS)])
        plsc.subcore_barrier()
        row0 = wid * RPT

        pltpu.sync_copy(ei_hbm.at[pl.ds(row0, KB)], idx.at[0])

        def group(g, carry):
            b = lax.rem(g, 2)
            bp = 1 - b
            for j in range(KB):
                pltpu.async_copy(tab_hbm.at[idx.at[b, j, 0]], rows.at[b, j], gsem)
            @pl.when(g >= 1)
            def _():
                for j in range(KB):
                    pltpu.make_async_copy(
                        rows.at[bp, j], acc.at[idx.at[bp, j, 1]], ssem
                    ).wait()
            @pl.when(g + 1 < G)
            def _():
                pltpu.async_copy(
                    ei_hbm.at[pl.ds(row0 + (g + 1) * KB, KB)], idx.at[bp], isem
                )
            for j in range(KB):
                pltpu.make_async_copy(tab_hbm.at[idx.at[b, j, 0]], rows.at[b, j], gsem).wait()
                pltpu.async_copy(rows.at[b, j], acc.at[idx.at[b, j, 1]], ssem, add=True)
            @pl.when(g + 1 < G)
            def _():
                pltpu.make_async_copy(
                    ei_hbm.at[pl.ds(row0 + (g + 1) * KB, KB)], idx.at[bp], isem
                ).wait()
            return carry

        lax.fori_loop(0, G, group, 0)
        bl = (G - 1) % 2
        for j in range(KB):
            pltpu.make_async_copy(rows.at[bl, j], acc.at[idx.at[bl, j, 1]], ssem).wait()
        plsc.subcore_barrier()
        pltpu.sync_copy(acc.at[pl.ds(s * NTS, NTS)], out_hbm.at[c, pl.ds(s * NTS, NTS)])

    return pl.kernel(
        body,
        out_type=jax.ShapeDtypeStruct((NC, NT, W), jnp.float32),
        mesh=mesh,
        compiler_params=pltpu.CompilerParams(use_tc_tiling_on_sc=False),
        scratch_types=[
            pltpu.VMEM((2, KB, 2, SUB), jnp.int32),
            pltpu.VMEM((2, KB, SUB, W), jnp.float32),
            pltpu.VMEM_SHARED((NT, W), jnp.float32),
            pltpu.SemaphoreType.DMA,
            pltpu.SemaphoreType.DMA,
            pltpu.SemaphoreType.DMA,
        ],
    )


@functools.lru_cache(maxsize=None)
def _make_sc_deg(NC, NS, R2, NT, RPT, G):
    mesh = plsc.VectorSubcoreMesh(core_axis_name="c", subcore_axis_name="s", num_cores=NC, num_subcores=NS)

    def body(dst_hbm, zero_hbm, out_hbm, idx, hist, isem):
        c = lax.axis_index("c")
        s = lax.axis_index("s")
        wid = s * NC + c
        pltpu.sync_copy(zero_hbm, hist)
        row0 = wid * RPT
        ones = jnp.ones((16,), jnp.float32)

        pltpu.sync_copy(dst_hbm.at[pl.ds(row0, KB)], idx.at[0])

        def group(g, carry):
            b = lax.rem(g, 2)
            bn = 1 - b
            @pl.when(g + 1 < G)
            def _():
                pltpu.async_copy(
                    dst_hbm.at[pl.ds(row0 + (g + 1) * KB, KB)], idx.at[bn], isem
                )
            for j in range(KB):
                for k in range(SUB // 16):
                    v = idx[b, j, pl.ds(k * 16, 16)]
                    plsc.addupdate_scatter(hist, [v], ones)
            @pl.when(g + 1 < G)
            def _():
                pltpu.make_async_copy(
                    dst_hbm.at[pl.ds(row0 + (g + 1) * KB, KB)], idx.at[bn], isem
                ).wait()
            return carry

        lax.fori_loop(0, G, group, 0)
        pltpu.sync_copy(hist, out_hbm.at[c, s])

    return pl.kernel(
        body,
        out_type=jax.ShapeDtypeStruct((NC, NS, NT), jnp.float32),
        mesh=mesh,
        compiler_params=pltpu.CompilerParams(
            use_tc_tiling_on_sc=False, needs_layout_passes=False
        ),
        scratch_types=[
            pltpu.VMEM((2, KB, SUB), jnp.int32),
            pltpu.VMEM((NT,), jnp.float32),
            pltpu.SemaphoreType.DMA,
        ],
    )


def _tc_prescale(degp, x_pad, BR):
    P, NT = degp.shape
    Din = x_pad.shape[1]

    def body(degp_ref, x_ref, ys_ref, dinv_ref):
        d = jnp.sum(degp_ref[...], axis=0)
        di = lax.rsqrt(d + 1.0)[:, None]
        dinv_ref[...] = jnp.broadcast_to(di, (BR, Din))
        ys_ref[...] = x_ref[...] * di

    return pl.pallas_call(
        body,
        grid=(NT // BR,),
        in_specs=[
            pl.BlockSpec((P, BR), lambda r: (0, r)),
            pl.BlockSpec((BR, Din), lambda r: (r, 0)),
        ],
        out_specs=[
            pl.BlockSpec((BR, Din), lambda r: (r, 0)),
            pl.BlockSpec((BR, Din), lambda r: (r, 0)),
        ],
        out_shape=[
            jax.ShapeDtypeStruct((NT, Din), jnp.float32),
            jax.ShapeDtypeStruct((NT, Din), jnp.float32),
        ],
    )(degp, x_pad)


def _tc_mid(S1p, ys1, dinv, W1, b1, W2, BR):
    NC, NT, Din = S1p.shape
    Dh = W1.shape[1]
    Do = W2.shape[1]

    def body(sp_ref, ys_ref, di_ref, w1_ref, b1_ref, w2_ref, out_ref):
        S = ys_ref[...]
        for i in range(NC):
            S = S + sp_ref[i]
        agg = di_ref[...] * S
        h = jnp.dot(agg, w1_ref[...], preferred_element_type=jnp.float32) + b1_ref[...]
        h = jnp.maximum(h, 0.0)
        y2 = jnp.dot(h, w2_ref[...], preferred_element_type=jnp.float32)
        out_ref[...] = y2 * di_ref[...]

    return pl.pallas_call(
        body,
        grid=(NT // BR,),
        in_specs=[
            pl.BlockSpec((NC, BR, Din), lambda r: (0, r, 0)),
            pl.BlockSpec((BR, Din), lambda r: (r, 0)),
            pl.BlockSpec((BR, Din), lambda r: (r, 0)),
            pl.BlockSpec((Din, Dh), lambda r: (0, 0)),
            pl.BlockSpec((1, Dh), lambda r: (0, 0)),
            pl.BlockSpec((Dh, Do), lambda r: (0, 0)),
        ],
        out_specs=pl.BlockSpec((BR, Do), lambda r: (r, 0)),
        out_shape=jax.ShapeDtypeStruct((NT, Do), jnp.float32),
    )(S1p, ys1, dinv, W1, b1.reshape(1, Dh), W2)


def _tc_final(S2p, ys2, dinv, b2, BR):
    NC, NT, Do = S2p.shape

    def body(sp_ref, ys_ref, di_ref, b2_ref, out_ref):
        S = ys_ref[...]
        for i in range(NC):
            S = S + sp_ref[i]
        out_ref[...] = di_ref[...] * S + b2_ref[...]

    return pl.pallas_call(
        body,
        grid=(NT // BR,),
        in_specs=[
            pl.BlockSpec((NC, BR, Do), lambda r: (0, r, 0)),
            pl.BlockSpec((BR, Do), lambda r: (r, 0)),
            pl.BlockSpec((BR, Do), lambda r: (r, 0)),
            pl.BlockSpec((1, Do), lambda r: (0, 0)),
        ],
        out_specs=pl.BlockSpec((BR, Do), lambda r: (r, 0)),
        out_shape=jax.ShapeDtypeStruct((NT, Do), jnp.float32),
    )(S2p, ys2, dinv, b2.reshape(1, Do))


def kernel(x, edge_index, W1, b1, W2, b2):
    N, Din = x.shape
    E = edge_index.shape[1]
    Dh = W1.shape[1]
    Do = W2.shape[1]
    NC, NS = _sc_info()
    NW = NC * NS

    unit = NW * SUB * KB
    E_pad = _round_up(E, unit)
    EPT = E_pad // NW
    RPT = EPT // SUB
    R2 = E_pad // SUB
    G = EPT // (SUB * KB)
    BR = 1024
    NT = _round_up(N + 1, BR)

    src = edge_index[0]
    dst = edge_index[1]
    padv = jnp.full((E_pad - E,), N, jnp.int32)
    srcp = jnp.concatenate([src, padv]).reshape(R2, SUB)
    dstp = jnp.concatenate([dst, padv]).reshape(R2, SUB)
    ei2 = jnp.stack([srcp, dstp], axis=1)
    x_pad = jnp.zeros((NT, Din), jnp.float32).at[:N].set(x)

    zeroN = jnp.zeros((NT,), jnp.float32)
    zero1 = jnp.zeros((NT, Din), jnp.float32)
    zero2 = jnp.zeros((NT, Do), jnp.float32)

    degp = _make_sc_deg(NC, NS, R2, NT, RPT, G)(dstp, zeroN)
    ys1, dinv = _tc_prescale(degp.reshape(NC * NS, NT), x_pad, BR)
    S1p = _make_sc_agg(NC, NS, R2, NT, Din, RPT, G)(ei2, ys1, zero1)
    ys2 = _tc_mid(S1p, ys1, dinv, W1, b1, W2, BR)
    S2p = _make_sc_agg(NC, NS, R2, NT, Do, RPT, G)(ei2, ys2, zero2)
    out = _tc_final(S2p, ys2, dinv, b2, BR)
    return out[:N]


# speedup vs baseline: 2.3983x; 1.0074x over previous
import functools

import jax
import jax.numpy as jnp
from jax import lax
from jax.experimental import pallas as pl
from jax.experimental.pallas import tpu as pltpu
from jax.experimental.pallas import tpu_sc as plsc

SUB = 128
KB = 6


def _sc_info():
    try:
        info = plsc.get_sparse_core_info()
        return info.num_cores, info.num_subcores
    except Exception:
        return 2, 16


def _round_up(a, b):
    return (a + b - 1) // b * b


@functools.lru_cache(maxsize=None)
def _make_sc_agg(NC, NS, R2, NT, W, RPT, G):
    mesh = plsc.VectorSubcoreMesh(core_axis_name="c", subcore_axis_name="s", num_cores=NC, num_subcores=NS)
    NTS = NT // NS

    PW = NT * W // 128
    PWS = PW // NS

    def body(ei_hbm, tab_hbm, zero_hbm, out_hbm, idx, rows, acc, gsem, ssem, isem):
        c = lax.axis_index("c")
        s = lax.axis_index("s")
        wid = s * NC + c
        pltpu.sync_copy(zero_hbm.at[pl.ds(s * NTS, NTS)], acc.at[pl.ds(s * NTS, NTS)])
        plsc.subcore_barrier()
        row0 = wid * RPT

        pltpu.sync_copy(ei_hbm.at[pl.ds(row0, KB)], idx.at[0])

        def group(g, carry):
            b = lax.rem(g, 2)
            bp = 1 - b
            for j in range(KB):
                pltpu.async_copy(tab_hbm.at[idx.at[b, j, 0]], rows.at[b, j], gsem)
            @pl.when(g >= 1)
            def _():
                for j in range(KB):
                    pltpu.make_async_copy(
                        rows.at[bp, j], acc.at[idx.at[bp, j, 1]], ssem
                    ).wait()
            @pl.when(g + 1 < G)
            def _():
                pltpu.async_copy(
                    ei_hbm.at[pl.ds(row0 + (g + 1) * KB, KB)], idx.at[bp], isem
                )
            for j in range(KB):
                pltpu.make_async_copy(tab_hbm.at[idx.at[b, j, 0]], rows.at[b, j], gsem).wait()
                pltpu.async_copy(rows.at[b, j], acc.at[idx.at[b, j, 1]], ssem, add=True)
            @pl.when(g + 1 < G)
            def _():
                pltpu.make_async_copy(
                    ei_hbm.at[pl.ds(row0 + (g + 1) * KB, KB)], idx.at[bp], isem
                ).wait()
            return carry

        lax.fori_loop(0, G, group, 0)
        bl = (G - 1) % 2
        for j in range(KB):
            pltpu.make_async_copy(rows.at[bl, j], acc.at[idx.at[bl, j, 1]], ssem).wait()
        plsc.subcore_barrier()
        pltpu.sync_copy(acc.at[pl.ds(s * NTS, NTS)], out_hbm.at[c, pl.ds(s * NTS, NTS)])

    return pl.kernel(
        body,
        out_type=jax.ShapeDtypeStruct((NC, NT, W), jnp.float32),
        mesh=mesh,
        compiler_params=pltpu.CompilerParams(use_tc_tiling_on_sc=False),
        scratch_types=[
            pltpu.VMEM((2, KB, 2, SUB), jnp.int32),
            pltpu.VMEM((2, KB, SUB, W), jnp.float32),
            pltpu.VMEM_SHARED((NT, W), jnp.float32),
            pltpu.SemaphoreType.DMA,
            pltpu.SemaphoreType.DMA,
            pltpu.SemaphoreType.DMA,
        ],
    )


@functools.lru_cache(maxsize=None)
def _make_sc_deg(NC, NS, R2, NT, RPT, G):
    mesh = plsc.VectorSubcoreMesh(core_axis_name="c", subcore_axis_name="s", num_cores=NC, num_subcores=NS)

    def body(dst_hbm, zero_hbm, out_hbm, idx, hist, isem):
        c = lax.axis_index("c")
        s = lax.axis_index("s")
        wid = s * NC + c
        pltpu.sync_copy(zero_hbm, hist)
        row0 = wid * RPT
        ones = jnp.ones((16,), jnp.float32)

        pltpu.sync_copy(dst_hbm.at[pl.ds(row0, KB)], idx.at[0])

        def group(g, carry):
            b = lax.rem(g, 2)
            bn = 1 - b
            @pl.when(g + 1 < G)
            def _():
                pltpu.async_copy(
                    dst_hbm.at[pl.ds(row0 + (g + 1) * KB, KB)], idx.at[bn], isem
                )
            for j in range(KB):
                for k in range(SUB // 16):
                    v = idx[b, j, pl.ds(k * 16, 16)]
                    plsc.addupdate_scatter(
                        hist, [jax.lax.shift_right_logical(v, 7), jnp.bitwise_and(v, 127)], ones
                    )
            @pl.when(g + 1 < G)
            def _():
                pltpu.make_async_copy(
                    dst_hbm.at[pl.ds(row0 + (g + 1) * KB, KB)], idx.at[bn], isem
                ).wait()
            return carry

        lax.fori_loop(0, G, group, 0)
        pltpu.sync_copy(hist, out_hbm.at[c, s])

    return pl.kernel(
        body,
        out_type=jax.ShapeDtypeStruct((NC, NS, NT // 128, 128), jnp.float32),
        mesh=mesh,
        compiler_params=pltpu.CompilerParams(
            use_tc_tiling_on_sc=False, needs_layout_passes=False
        ),
        scratch_types=[
            pltpu.VMEM((2, KB, SUB), jnp.int32),
            pltpu.VMEM((NT // 128, 128), jnp.float32),
            pltpu.SemaphoreType.DMA,
        ],
    )


def _tc_prescale(degp, x_pad, BR):
    P, NB, _ = degp.shape
    NT = NB * 128
    Din = x_pad.shape[1]
    BB = BR // 128

    def body(degp_ref, x_ref, ys_ref, dinv_ref):
        d = jnp.sum(degp_ref[...], axis=0).reshape(BR)
        di = lax.rsqrt(d + 1.0)[:, None]
        dinv_ref[...] = jnp.broadcast_to(di, (BR, Din))
        ys_ref[...] = x_ref[...] * di

    return pl.pallas_call(
        body,
        grid=(NT // BR,),
        in_specs=[
            pl.BlockSpec((P, BB, 128), lambda r: (0, r, 0)),
            pl.BlockSpec((BR, Din), lambda r: (r, 0)),
        ],
        out_specs=[
            pl.BlockSpec((BR, Din), lambda r: (r, 0)),
            pl.BlockSpec((BR, Din), lambda r: (r, 0)),
        ],
        out_shape=[
            jax.ShapeDtypeStruct((NT, Din), jnp.float32),
            jax.ShapeDtypeStruct((NT, Din), jnp.float32),
        ],
    )(degp, x_pad)


def _tc_mid(S1p, ys1, dinv, W1, b1, W2, BR):
    NC = S1p.shape[0]
    NT, Din = ys1.shape
    Dh = W1.shape[1]
    Do = W2.shape[1]
    BPW = BR * Din // 128

    def body(sp_ref, ys_ref, di_ref, w1_ref, b1_ref, w2_ref, out_ref):
        S = ys_ref[...]
        for i in range(NC):
            S = S + sp_ref[i]
        agg = di_ref[...] * S
        h = jnp.dot(agg, w1_ref[...], preferred_element_type=jnp.float32) + b1_ref[...]
        h = jnp.maximum(h, 0.0)
        y2 = jnp.dot(h, w2_ref[...], preferred_element_type=jnp.float32)
        out_ref[...] = y2 * di_ref[...]

    return pl.pallas_call(
        body,
        grid=(NT // BR,),
        in_specs=[
            pl.BlockSpec((NC, BR, Din), lambda r: (0, r, 0)),
            pl.BlockSpec((BR, Din), lambda r: (r, 0)),
            pl.BlockSpec((BR, Din), lambda r: (r, 0)),
            pl.BlockSpec((Din, Dh), lambda r: (0, 0)),
            pl.BlockSpec((1, Dh), lambda r: (0, 0)),
            pl.BlockSpec((Dh, Do), lambda r: (0, 0)),
        ],
        out_specs=pl.BlockSpec((BR, Do), lambda r: (r, 0)),
        out_shape=jax.ShapeDtypeStruct((NT, Do), jnp.float32),
    )(S1p, ys1, dinv, W1, b1.reshape(1, Dh), W2)


def _tc_final(S2p, ys2, dinv, b2, BR):
    NC = S2p.shape[0]
    NT, Do = ys2.shape
    BPW = BR * Do // 128

    def body(sp_ref, ys_ref, di_ref, b2_ref, out_ref):
        S = ys_ref[...]
        for i in range(NC):
            S = S + sp_ref[i]
        out_ref[...] = di_ref[...] * S + b2_ref[...]

    return pl.pallas_call(
        body,
        grid=(NT // BR,),
        in_specs=[
            pl.BlockSpec((NC, BR, Do), lambda r: (0, r, 0)),
            pl.BlockSpec((BR, Do), lambda r: (r, 0)),
            pl.BlockSpec((BR, Do), lambda r: (r, 0)),
            pl.BlockSpec((1, Do), lambda r: (0, 0)),
        ],
        out_specs=pl.BlockSpec((BR, Do), lambda r: (r, 0)),
        out_shape=jax.ShapeDtypeStruct((NT, Do), jnp.float32),
    )(S2p, ys2, dinv, b2.reshape(1, Do))


def kernel(x, edge_index, W1, b1, W2, b2):
    N, Din = x.shape
    E = edge_index.shape[1]
    Dh = W1.shape[1]
    Do = W2.shape[1]
    NC, NS = _sc_info()
    NW = NC * NS

    unit = NW * SUB * KB
    E_pad = _round_up(E, unit)
    EPT = E_pad // NW
    RPT = EPT // SUB
    R2 = E_pad // SUB
    G = EPT // (SUB * KB)
    BR = 1024
    NT = _round_up(N + 1, BR)

    src = edge_index[0]
    dst = edge_index[1]
    padv = jnp.full((E_pad - E,), N, jnp.int32)
    srcp = jnp.concatenate([src, padv]).reshape(R2, SUB)
    dstp = jnp.concatenate([dst, padv]).reshape(R2, SUB)
    ei2 = jnp.stack([srcp, dstp], axis=1)
    x_pad = jnp.zeros((NT, Din), jnp.float32).at[:N].set(x)

    zeroN = jnp.zeros((NT // 128, 128), jnp.float32)
    zero1 = jnp.zeros((NT, Din), jnp.float32)
    zero2 = jnp.zeros((NT, Do), jnp.float32)

    degp = _make_sc_deg(NC, NS, R2, NT, RPT, G)(dstp, zeroN)
    ys1, dinv = _tc_prescale(degp.reshape(NC * NS, NT // 128, 128), x_pad, BR)
    S1p = _make_sc_agg(NC, NS, R2, NT, Din, RPT, G)(ei2, ys1, zero1)
    ys2 = _tc_mid(S1p, ys1, dinv, W1, b1, W2, BR)
    S2p = _make_sc_agg(NC, NS, R2, NT, Do, RPT, G)(ei2, ys2, zero2)
    out = _tc_final(S2p, ys2, dinv, b2, BR)
    return out[:N]


# speedup vs baseline: 2.3983x; 1.0000x over previous
import functools

import jax
import jax.numpy as jnp
from jax import lax
from jax.experimental import pallas as pl
from jax.experimental.pallas import tpu as pltpu
from jax.experimental.pallas import tpu_sc as plsc

SUB = 128
KB = 6


def _sc_info():
    try:
        info = plsc.get_sparse_core_info()
        return info.num_cores, info.num_subcores
    except Exception:
        return 2, 16


def _round_up(a, b):
    return (a + b - 1) // b * b


@functools.lru_cache(maxsize=None)
def _make_sc_agg(NC, NS, R2, NT, W, RPT, G):
    mesh = plsc.VectorSubcoreMesh(core_axis_name="c", subcore_axis_name="s", num_cores=NC, num_subcores=NS)
    NTS = NT // NS

    def body(ei_hbm, tab_hbm, zero_hbm, out_hbm, idx, rows, acc, gsem, ssem, isem):
        c = lax.axis_index("c")
        s = lax.axis_index("s")
        wid = s * NC + c
        pltpu.sync_copy(zero_hbm.at[pl.ds(s * NTS, NTS)], acc.at[pl.ds(s * NTS, NTS)])
        plsc.subcore_barrier()
        row0 = wid * RPT

        pltpu.sync_copy(ei_hbm.at[pl.ds(row0, KB)], idx.at[0])

        def group(g, carry):
            b = lax.rem(g, 2)
            bp = 1 - b
            for j in range(KB):
                pltpu.async_copy(tab_hbm.at[idx.at[b, j, 0]], rows.at[b, j], gsem)
            @pl.when(g >= 1)
            def _():
                for j in range(KB):
                    pltpu.make_async_copy(
                        rows.at[bp, j], acc.at[idx.at[bp, j, 1]], ssem
                    ).wait()
            @pl.when(g + 1 < G)
            def _():
                pltpu.async_copy(
                    ei_hbm.at[pl.ds(row0 + (g + 1) * KB, KB)], idx.at[bp], isem
                )
            for j in range(KB):
                pltpu.make_async_copy(tab_hbm.at[idx.at[b, j, 0]], rows.at[b, j], gsem).wait()
                pltpu.async_copy(rows.at[b, j], acc.at[idx.at[b, j, 1]], ssem, add=True)
            @pl.when(g + 1 < G)
            def _():
                pltpu.make_async_copy(
                    ei_hbm.at[pl.ds(row0 + (g + 1) * KB, KB)], idx.at[bp], isem
                ).wait()
            return carry

        lax.fori_loop(0, G, group, 0)
        bl = (G - 1) % 2
        for j in range(KB):
            pltpu.make_async_copy(rows.at[bl, j], acc.at[idx.at[bl, j, 1]], ssem).wait()
        plsc.subcore_barrier()
        pltpu.sync_copy(acc.at[pl.ds(s * NTS, NTS)], out_hbm.at[c, pl.ds(s * NTS, NTS)])

    return pl.kernel(
        body,
        out_type=jax.ShapeDtypeStruct((NC, NT, W), jnp.float32),
        mesh=mesh,
        compiler_params=pltpu.CompilerParams(use_tc_tiling_on_sc=False),
        scratch_types=[
            pltpu.VMEM((2, KB, 2, SUB), jnp.int32),
            pltpu.VMEM((2, KB, SUB, W), jnp.float32),
            pltpu.VMEM_SHARED((NT, W), jnp.float32),
            pltpu.SemaphoreType.DMA,
            pltpu.SemaphoreType.DMA,
            pltpu.SemaphoreType.DMA,
        ],
    )


@functools.lru_cache(maxsize=None)
def _make_sc_deg(NC, NS, R2, NT, RPT, G):
    mesh = plsc.VectorSubcoreMesh(core_axis_name="c", subcore_axis_name="s", num_cores=NC, num_subcores=NS)

    def body(dst_hbm, zero_hbm, out_hbm, idx, hist, isem):
        c = lax.axis_index("c")
        s = lax.axis_index("s")
        wid = s * NC + c
        pltpu.sync_copy(zero_hbm, hist)
        row0 = wid * RPT
        ones = jnp.ones((16,), jnp.float32)

        pltpu.sync_copy(dst_hbm.at[pl.ds(row0, KB)], idx.at[0])

        def group(g, carry):
            b = lax.rem(g, 2)
            bn = 1 - b
            @pl.when(g + 1 < G)
            def _():
                pltpu.async_copy(
                    dst_hbm.at[pl.ds(row0 + (g + 1) * KB, KB)], idx.at[bn], isem
                )
            for j in range(KB):
                for k in range(SUB // 16):
                    v = idx[b, j, pl.ds(k * 16, 16)]
                    plsc.addupdate_scatter(
                        hist, [jax.lax.shift_right_logical(v, 7), jnp.bitwise_and(v, 127)], ones
                    )
            @pl.when(g + 1 < G)
            def _():
                pltpu.make_async_copy(
                    dst_hbm.at[pl.ds(row0 + (g + 1) * KB, KB)], idx.at[bn], isem
                ).wait()
            return carry

        lax.fori_loop(0, G, group, 0)
        pltpu.sync_copy(hist, out_hbm.at[c, s])

    return pl.kernel(
        body,
        out_type=jax.ShapeDtypeStruct((NC, NS, NT // 128, 128), jnp.float32),
        mesh=mesh,
        compiler_params=pltpu.CompilerParams(
            use_tc_tiling_on_sc=False, needs_layout_passes=False
        ),
        scratch_types=[
            pltpu.VMEM((2, KB, SUB), jnp.int32),
            pltpu.VMEM((NT // 128, 128), jnp.float32),
            pltpu.SemaphoreType.DMA,
        ],
    )


def _tc_prescale(degp, x_pad, BR):
    P, NB, _ = degp.shape
    NT = NB * 128
    Din = x_pad.shape[1]
    BB = BR // 128

    def body(degp_ref, x_ref, ys_ref, dinv_ref):
        d = jnp.sum(degp_ref[...], axis=0).reshape(BR)
        di = lax.rsqrt(d + 1.0)[:, None]
        dinv_ref[...] = jnp.broadcast_to(di, (BR, Din))
        ys_ref[...] = x_ref[...] * di

    return pl.pallas_call(
        body,
        grid=(NT // BR,),
        in_specs=[
            pl.BlockSpec((P, BB, 128), lambda r: (0, r, 0)),
            pl.BlockSpec((BR, Din), lambda r: (r, 0)),
        ],
        out_specs=[
            pl.BlockSpec((BR, Din), lambda r: (r, 0)),
            pl.BlockSpec((BR, Din), lambda r: (r, 0)),
        ],
        out_shape=[
            jax.ShapeDtypeStruct((NT, Din), jnp.float32),
            jax.ShapeDtypeStruct((NT, Din), jnp.float32),
        ],
    )(degp, x_pad)


def _tc_mid(S1p, ys1, dinv, W1, b1, W2, BR):
    NC = S1p.shape[0]
    NT, Din = ys1.shape
    Dh = W1.shape[1]
    Do = W2.shape[1]

    def body(sp_ref, ys_ref, di_ref, w1_ref, b1_ref, w2_ref, out_ref):
        S = ys_ref[...]
        for i in range(NC):
            S = S + sp_ref[i]
        agg = di_ref[...] * S
        h = jnp.dot(agg, w1_ref[...], preferred_element_type=jnp.float32) + b1_ref[...]
        h = jnp.maximum(h, 0.0)
        y2 = jnp.dot(h, w2_ref[...], preferred_element_type=jnp.float32)
        out_ref[...] = y2 * di_ref[...]

    return pl.pallas_call(
        body,
        grid=(NT // BR,),
        in_specs=[
            pl.BlockSpec((NC, BR, Din), lambda r: (0, r, 0)),
            pl.BlockSpec((BR, Din), lambda r: (r, 0)),
            pl.BlockSpec((BR, Din), lambda r: (r, 0)),
            pl.BlockSpec((Din, Dh), lambda r: (0, 0)),
            pl.BlockSpec((1, Dh), lambda r: (0, 0)),
            pl.BlockSpec((Dh, Do), lambda r: (0, 0)),
        ],
        out_specs=pl.BlockSpec((BR, Do), lambda r: (r, 0)),
        out_shape=jax.ShapeDtypeStruct((NT, Do), jnp.float32),
    )(S1p, ys1, dinv, W1, b1.reshape(1, Dh), W2)


def _tc_final(S2p, ys2, dinv, b2, BR):
    NC = S2p.shape[0]
    NT, Do = ys2.shape

    def body(sp_ref, ys_ref, di_ref, b2_ref, out_ref):
        S = ys_ref[...]
        for i in range(NC):
            S = S + sp_ref[i]
        out_ref[...] = di_ref[...] * S + b2_ref[...]

    return pl.pallas_call(
        body,
        grid=(NT // BR,),
        in_specs=[
            pl.BlockSpec((NC, BR, Do), lambda r: (0, r, 0)),
            pl.BlockSpec((BR, Do), lambda r: (r, 0)),
            pl.BlockSpec((BR, Do), lambda r: (r, 0)),
            pl.BlockSpec((1, Do), lambda r: (0, 0)),
        ],
        out_specs=pl.BlockSpec((BR, Do), lambda r: (r, 0)),
        out_shape=jax.ShapeDtypeStruct((NT, Do), jnp.float32),
    )(S2p, ys2, dinv, b2.reshape(1, Do))


def kernel(x, edge_index, W1, b1, W2, b2):
    N, Din = x.shape
    E = edge_index.shape[1]
    Dh = W1.shape[1]
    Do = W2.shape[1]
    NC, NS = _sc_info()
    NW = NC * NS

    unit = NW * SUB * KB
    E_pad = _round_up(E, unit)
    EPT = E_pad // NW
    RPT = EPT // SUB
    R2 = E_pad // SUB
    G = EPT // (SUB * KB)
    BR = 1024
    NT = _round_up(N + 1, BR)

    src = edge_index[0]
    dst = edge_index[1]
    padv = jnp.full((E_pad - E,), N, jnp.int32)
    srcp = jnp.concatenate([src, padv]).reshape(R2, SUB)
    dstp = jnp.concatenate([dst, padv]).reshape(R2, SUB)
    ei2 = jnp.stack([srcp, dstp], axis=1)
    x_pad = jnp.zeros((NT, Din), jnp.float32).at[:N].set(x)

    zeroN = jnp.zeros((NT // 128, 128), jnp.float32)
    zero1 = jnp.zeros((NT, Din), jnp.float32)
    zero2 = jnp.zeros((NT, Do), jnp.float32)

    degp = _make_sc_deg(NC, NS, R2, NT, RPT, G)(dstp, zeroN)
    ys1, dinv = _tc_prescale(degp.reshape(NC * NS, NT // 128, 128), x_pad, BR)
    S1p = _make_sc_agg(NC, NS, R2, NT, Din, RPT, G)(ei2, ys1, zero1)
    ys2 = _tc_mid(S1p, ys1, dinv, W1, b1, W2, BR)
    S2p = _make_sc_agg(NC, NS, R2, NT, Do, RPT, G)(ei2, ys2, zero2)
    out = _tc_final(S2p, ys2, dinv, b2, BR)
    return out[:N]
